# baseline (speedup 1.0000x reference)
import functools
import jax
import jax.numpy as jnp
from jax import lax
from jax.experimental import pallas as pl
from jax.experimental.pallas import tpu as pltpu
from jax.experimental.pallas import tpu_sc as plsc

N = 10000
E = 160000
B = 4
D = 64
INPUT_DIM = 64
PRE_K = 4
PRE_V = 2

NC, NS, L = 2, 16, 16
IS = 192
FC = 128
NCHUNK = 6
CPS = NCHUNK // NC
CN6 = NCHUNK * N
EP = E // NS
EPP = 10240
KB = 32
NBUF = 5
GS = NBUF * KB
NOG = EPP // GS
NSE = NS * EPP
WB = 40
RB = 640
RBLK = 80
FCV = FC // L
KV = KB // L


def _scale_rows(buf, vv, b):
    @plsc.parallel_loop(0, KV)
    def _(j):
        vvv = vv[b, pl.ds(j * L, L)]
        for l in range(L):
            v = jnp.full((L,), vvv[l], jnp.float32)
            r = j * L + l
            for k in range(FCV):
                buf[r, pl.ds(k * L, L)] = buf[r, pl.ds(k * L, L)] * v


def _sc_body(x0_hbm, rc_hbm, vals_hbm, ys_hbm,
             acc, degs,
             giA, giB, siA, siB, ajA, ajB,
             gxA, gxB, dxA, dxB, sxA, sxB, dvA, dvB,
             gbuf, zb, rb,
             esemA, esemB, dsems, gsems, ssems, wsems):
    sets = {0: (giA, siA, ajA, gxA, dxA, sxA, dvA),
            1: (giB, siB, ajB, gxB, dxB, sxB, dvB)}
    cid = lax.axis_index("c")
    sid = lax.axis_index("s")
    ebase = sid * EPP
    nblk2 = jnp.where(sid == NS - 1, 5, 8)
    nrb = jnp.where(sid == NS - 1, 5, 8)

    def zero_zb():
        @plsc.parallel_loop(0, WB)
        def _(r):
            for k in range(FCV):
                zb[r, pl.ds(k * L, L)] = jnp.zeros((L,), jnp.float32)

    zero_zb()
    for i in range(RBLK // L):
        rb[pl.ds(i * L, L)] = jnp.zeros((L,), jnp.float32)

    def dz_body(w, _):
        r0 = sid * RB + w * RBLK
        pltpu.sync_copy(rb, degs.at[pl.ds(r0, RBLK)])
        pltpu.sync_copy(rb, degs.at[pl.ds(N + r0, RBLK)])
        return 0
    lax.fori_loop(0, nrb, dz_body, 0)

    def x0q(q, _):
        cn = (cid * CPS + q) * N

        def x0w(w, _):
            r0 = pl.multiple_of(cn + sid * RB + w * WB, 40)
            pltpu.sync_copy(x0_hbm.at[pl.ds(r0, WB)], zb)
            pltpu.sync_copy(zb, ys_hbm.at[pl.ds(r0, WB)])
            return 0
        lax.fori_loop(0, 2 * nblk2, x0w, 0)
        return 0
    lax.fori_loop(0, CPS, x0q, 0)
    zero_zb()
    plsc.subcore_barrier()

    def deg_og(og, _):
        e0 = pl.multiple_of(ebase + og * GS, GS)
        el = [pltpu.async_copy(rc_hbm.at[pl.ds(e0, GS)], giA, esemA),
              pltpu.async_copy(rc_hbm.at[pl.ds(pl.multiple_of(NSE + e0, GS), GS)], siA, esemA),
              pltpu.async_copy(vals_hbm.at[pl.ds(e0, GS)], ajA, esemA)]
        for d in el:
            d.wait()
        sds = []
        for b in range(NBUF):
            for k in range(KV):
                sl2 = pl.ds(k * L, L)
                sl1 = pl.ds(b * KB + k * L, L)
                dxA[b, sl2] = giA[sl1]
                sxA[b, sl2] = siA[sl1] + N
                dvA[b, sl2] = ajA[sl1]
            sds.append(pltpu.async_copy(
                dvA.at[b], degs.at[dxA.at[b]], gsems[b], add=True))
            sds.append(pltpu.async_copy(
                dvA.at[b], degs.at[sxA.at[b]], ssems[b], add=True))
        for d in sds:
            d.wait()
        return 0
    lax.fori_loop(0, NOG, deg_og, 0)
    plsc.subcore_barrier()

    def rhalf(j, _):
        def rbod(w, _):
            r0 = pl.multiple_of(j * N + sid * RB + w * RBLK, RBLK)
            pltpu.sync_copy(degs.at[pl.ds(r0, RBLK)], rb)
            for i in range(RBLK // L):
                sl = pl.ds(i * L, L)
                v = rb[sl]
                rb[sl] = jnp.where(v > 0.0, 1.0 / jnp.where(v > 0.0, v, 1.0), 0.0)
            pltpu.sync_copy(rb, degs.at[pl.ds(r0, RBLK)])
            return 0
        lax.fori_loop(0, nrb, rbod, 0)
        return 0
    lax.fori_loop(0, 2, rhalf, 0)
    for i in range(RBLK // L):
        rb[pl.ds(i * L, L)] = jnp.zeros((L,), jnp.float32)
    plsc.subcore_barrier()

    def stage_step(t, _):
        st = t % 4
        q = t // 4
        cn = (cid * CPS + q) * N
        gsel = (st >= 2).astype(jnp.int32)
        src_slot = jnp.where(st == 0, 0, jnp.where(st == 3, 3, 1))
        goff = pl.multiple_of(gsel * NSE, GS)
        soff = pl.multiple_of((1 - gsel) * NSE, GS)
        dbase = gsel * N
        sbase = src_slot * CN6 + cn
        dstb = pl.multiple_of((st + 1) * CN6 + cn, WB)

        def load_edges(og_idx, S, sem):
            e0 = pl.multiple_of(ebase + og_idx * GS, GS)
            giS, siS, ajS = sets[S][0], sets[S][1], sets[S][2]
            return [pltpu.async_copy(rc_hbm.at[pl.ds(pl.multiple_of(goff + e0, GS), GS)], giS, sem),
                    pltpu.async_copy(rc_hbm.at[pl.ds(pl.multiple_of(soff + e0, GS), GS)], siS, sem),
                    pltpu.async_copy(vals_hbm.at[pl.ds(e0, GS)], ajS, sem)]

        def drain_edges(og_idx, S, sem):
            e0 = pl.multiple_of(ebase + og_idx * GS, GS)
            giS, siS, ajS = sets[S][0], sets[S][1], sets[S][2]
            pltpu.make_async_copy(rc_hbm.at[pl.ds(pl.multiple_of(goff + e0, GS), GS)], giS, sem).wait()
            pltpu.make_async_copy(rc_hbm.at[pl.ds(pl.multiple_of(soff + e0, GS), GS)], siS, sem).wait()
            pltpu.make_async_copy(vals_hbm.at[pl.ds(e0, GS)], ajS, sem).wait()

        def build(S):
            giS, siS, _, gxS, dxS, sxS, _ = sets[S]
            for b in range(NBUF):
                for k in range(KV):
                    sl2 = pl.ds(k * L, L)
                    sl1 = pl.ds(b * KB + k * L, L)
                    giv = giS[sl1]
                    gxS[b, sl2] = giv + sbase
                    dxS[b, sl2] = giv + dbase
                    sxS[b, sl2] = siS[sl1]

        def fire_rg(S):
            _, _, _, gxS, dxS, _, dvS = sets[S]
            for b in range(NBUF):
                pltpu.async_copy(degs.at[dxS.at[b]], dvS.at[b], dsems[b])
                pltpu.async_copy(ys_hbm.at[gxS.at[b]], gbuf.at[b], gsems[b])

        def process(S):
            _, _, ajS, gxS, dxS, sxS, dvS = sets[S]
            for b in range(NBUF):
                pltpu.make_async_copy(degs.at[dxS.at[b]], dvS.at[b], dsems[b]).wait()
                pltpu.make_async_copy(ys_hbm.at[gxS.at[b]], gbuf.at[b], gsems[b]).wait()
                for k in range(KV):
                    dvS[b, pl.ds(k * L, L)] = (
                        dvS[b, pl.ds(k * L, L)] * ajS[pl.ds(b * KB + k * L, L)])
                _scale_rows(gbuf.at[b], dvS, b)
                pltpu.async_copy(gbuf.at[b], acc.at[sxS.at[b]], ssems[b], add=True)

        def drain_sc(S):
            sxS = sets[S][5]
            for b in range(NBUF):
                pltpu.make_async_copy(gbuf.at[b], acc.at[sxS.at[b]], ssems[b]).wait()

        def sub(S, og, semc, semn):
            process(S)
            drain_edges(og + 1, 1 - S, semn)
            build(1 - S)
            load_edges(jnp.minimum(og + 2, NOG - 1), S, semc)
            drain_sc(S)
            fire_rg(1 - S)

        def zbod(w, _):
            pltpu.sync_copy(zb, acc.at[pl.ds(sid * RB + w * WB, WB)])
            return 0
        lax.fori_loop(0, 2 * nblk2, zbod, 0)
        plsc.subcore_barrier()

        for d in load_edges(0, 0, esemA):
            d.wait()
        build(0)
        fire_rg(0)
        load_edges(1, 1, esemB)

        def hbody(h, _):
            sub(0, 2 * h, esemA, esemB)
            sub(1, 2 * h + 1, esemB, esemA)
            return 0
        lax.fori_loop(0, NOG // 2 - 1, hbody, 0)
        sub(0, NOG - 2, esemA, esemB)
        process(1)
        drain_edges(NOG - 1, 0, esemA)
        drain_sc(1)
        plsc.subcore_barrier()

        def wbod(w, _):
            ds0 = []
            for j in range(2):
                r0 = sid * RB + (w * 2 + j) * WB
                ds0.append(pltpu.async_copy(
                    acc.at[pl.ds(r0, WB)], ys_hbm.at[pl.ds(pl.multiple_of(dstb + r0, WB), WB)], wsems[j]))
            for d in ds0:
                d.wait()
            return 0
        lax.fori_loop(0, nblk2, wbod, 0)
        plsc.subcore_barrier()
        return 0

    lax.fori_loop(0, 4 * CPS, stage_step, 0)


_sc_diffusion = functools.partial(
    pl.kernel,
    out_type=jax.ShapeDtypeStruct((5 * CN6, FC), jnp.float32),
    mesh=plsc.VectorSubcoreMesh(core_axis_name="c", subcore_axis_name="s"),
    compiler_params=pltpu.CompilerParams(needs_layout_passes=False),
    scratch_types=[
        pltpu.VMEM_SHARED((N, FC), jnp.float32),
        pltpu.VMEM_SHARED((2 * N,), jnp.float32),
        pltpu.VMEM((GS,), jnp.int32),
        pltpu.VMEM((GS,), jnp.int32),
        pltpu.VMEM((GS,), jnp.int32),
        pltpu.VMEM((GS,), jnp.int32),
        pltpu.VMEM((GS,), jnp.float32),
        pltpu.VMEM((GS,), jnp.float32),
        pltpu.VMEM((NBUF, KB), jnp.int32),
        pltpu.VMEM((NBUF, KB), jnp.int32),
        pltpu.VMEM((NBUF, KB), jnp.int32),
        pltpu.VMEM((NBUF, KB), jnp.int32),
        pltpu.VMEM((NBUF, KB), jnp.int32),
        pltpu.VMEM((NBUF, KB), jnp.int32),
        pltpu.VMEM((NBUF, KB), jnp.float32),
        pltpu.VMEM((NBUF, KB), jnp.float32),
        pltpu.VMEM((NBUF, KB, FC), jnp.float32),
        pltpu.VMEM((WB, FC), jnp.float32),
        pltpu.VMEM((RBLK,), jnp.float32),
        pltpu.SemaphoreType.DMA,
        pltpu.SemaphoreType.DMA,
        [pltpu.SemaphoreType.DMA] * NBUF,
        [pltpu.SemaphoreType.DMA] * NBUF,
        [pltpu.SemaphoreType.DMA] * NBUF,
        [pltpu.SemaphoreType.DMA] * 2,
    ],
)(_sc_body)


def _tail_body(conv_ref, w_ref, b_ref, att_ref, out_ref):
    c = conv_ref[...]
    c = jnp.where(c >= 0, c, 0.01 * c)
    out_ref[...] = jnp.dot(c, w_ref[...], preferred_element_type=jnp.float32) + b_ref[...] + att_ref[...]


def kernel(inputs, hx_k, adj_vals, rows, cols, gconv_w, gconv_b, W, b, R, att_w, att_b):
    preH = jnp.concatenate([hx_k[:, PRE_K - 1], hx_k[:, PRE_K - 2]], axis=-1)
    x = jnp.concatenate([inputs.reshape(B, N, INPUT_DIM), preH], axis=2)
    x0c = (x.transpose(1, 0, 2).reshape(N, NCHUNK, FC)
           .transpose(1, 0, 2).reshape(CN6, FC))
    pad = ((0, 0), (0, EPP - EP))
    rows1 = jnp.pad(rows.astype(jnp.int32).reshape(NS, EP), pad).reshape(-1)
    cols1 = jnp.pad(cols.astype(jnp.int32).reshape(NS, EP), pad).reshape(-1)
    vals1 = jnp.pad(adj_vals.reshape(NS, EP), pad).reshape(-1)
    rc1 = jnp.concatenate([rows1, cols1])

    ys = _sc_diffusion(x0c, rc1, vals1)
    y1c = ys[CN6:2 * CN6]
    s2c = ys[2 * CN6:3 * CN6]
    y3c = ys[3 * CN6:4 * CN6]
    s4c = ys[4 * CN6:5 * CN6]

    def unchunk(a):
        return (a.reshape(NCHUNK, N, FC).transpose(1, 0, 2)
                .reshape(N, B, IS).transpose(1, 0, 2))

    Wm = gconv_w.reshape(IS, 5, D)
    Weff = jnp.stack([Wm[:, 0] - Wm[:, 2], Wm[:, 1] - Wm[:, 4],
                      2.0 * Wm[:, 2], Wm[:, 3], 2.0 * Wm[:, 4]], axis=0)
    xs = jnp.stack([unchunk(x0c), unchunk(y1c), unchunk(s2c),
                    unchunk(y3c), unchunk(s4c)], axis=0)
    conv = jnp.einsum('mbni,mid->bnd', xs, Weff) + gconv_b

    new_states = hx_k + R[None]
    logits = jnp.matmul(new_states.reshape(B, PRE_K, N * D), att_w) + att_b
    weight = jax.nn.softmax(logits, axis=1)
    att = (new_states.reshape(B, PRE_K, N * D) * weight).sum(axis=1).reshape(B, N, D)

    BN = 400
    out = pl.pallas_call(
        _tail_body,
        grid=(B, N // BN),
        in_specs=[
            pl.BlockSpec((1, BN, D), lambda bb, nb: (bb, nb, 0)),
            pl.BlockSpec((D, D), lambda bb, nb: (0, 0)),
            pl.BlockSpec((BN, D), lambda bb, nb: (nb, 0)),
            pl.BlockSpec((1, BN, D), lambda bb, nb: (bb, nb, 0)),
        ],
        out_specs=pl.BlockSpec((1, BN, D), lambda bb, nb: (bb, nb, 0)),
        out_shape=jax.ShapeDtypeStruct((B, N, D), jnp.float32),
    )(conv, W, b, att)

    hx_new = jnp.concatenate([hx_k[:, 1:PRE_K], out[:, None]], axis=1)
    return out.reshape(B, N * D), hx_new
